# Initial kernel scaffold; baseline (speedup 1.0000x reference)
import jax
import jax.numpy as jnp
from jax.experimental import pallas as pl


def kernel(x, pos, edge_index, edge_attr, batch, W1, W2, W3, W4, W5, W6, W7, g1, g2, g3, g4, g5, g6, g7, b1, b2, b3, b4, b5, b6, b7, fcW):
    raise NotImplementedError("write your pallas kernel here")


# speedup vs baseline: 15.7140x; 15.7140x over previous
import functools

import jax
import jax.numpy as jnp
import numpy as np
from jax import lax
from jax.experimental import pallas as pl
from jax.experimental.pallas import tpu as pltpu
from jax.experimental.pallas import tpu_sc as plsc

_L = 16
_CHUNK = 64
_NC = 2
_NS = 16
_ROW = 128


def _rup(v, m):
    return (v + m - 1) // m * m


def _make_conv(n_chunks, NRh, half, interpret=False):
    rows16 = NRh // _NS
    nzc = rows16 // 32
    dump = NRh - 1
    mesh = plsc.VectorSubcoreMesh(core_axis_name="c", subcore_axis_name="s")

    @functools.partial(
        pl.kernel,
        mesh=mesh,
        out_type=jax.ShapeDtypeStruct((_NC * NRh, _ROW), jnp.float32),
        scratch_types=[
            pltpu.VMEM_SHARED((NRh, _ROW), jnp.float32),
            pltpu.VMEM((_CHUNK,), jnp.int32),
            pltpu.VMEM((1, _CHUNK), jnp.int32),
            pltpu.VMEM((1, _CHUNK), jnp.int32),
            pltpu.VMEM((_CHUNK, _L), jnp.float32),
            pltpu.VMEM((_CHUNK, _ROW), jnp.float32),
            pltpu.VMEM((_CHUNK, _ROW), jnp.float32),
            pltpu.SemaphoreType.DMA,
        ],
        interpret=interpret,
    )
    def k(tbl_hbm, src_hbm, dst_hbm, bas_hbm, zrow_hbm, out_hbm,
          acc_sh, srcv, dstv, dstloc, basv, rows, msg, sem):
        c = lax.axis_index("c")
        s = lax.axis_index("s")

        for z in range(nzc):
            pltpu.sync_copy(zrow_hbm,
                            acc_sh.at[pl.ds(s * rows16 + z * 32, 32)])
        plsc.subcore_barrier()

        def edge(e, carry):
            brow = basv[e, pl.ds(0, _L)]
            xrow = rows[e, pl.ds(0, _L)]
            for b in range(8):
                bb = brow.at[jnp.full((_L,), b, jnp.int32)].get(
                    mode="promise_in_bounds")
                msg[e, pl.ds(b * _L, _L)] = bb * xrow
            return carry

        def chunk(j, carry):
            e0 = (s * n_chunks + j) * _CHUNK
            pltpu.sync_copy(src_hbm.at[pl.ds(e0, _CHUNK)], srcv)
            pltpu.sync_copy(dst_hbm.at[pl.ds(e0, _CHUNK)],
                            dstv.at[0])
            pltpu.sync_copy(bas_hbm.at[pl.ds(e0, _CHUNK)], basv)
            pltpu.async_copy(tbl_hbm.at[srcv], rows, sem).wait()
            for g in range(_CHUNK // _L):
                d16 = dstv[0, pl.ds(g * _L, _L)] - c * half
                ok = (d16 >= 0) & (d16 < half)
                dstloc[0, pl.ds(g * _L, _L)] = jnp.where(ok, d16, dump)
            lax.fori_loop(0, _CHUNK, edge, 0)
            pltpu.sync_copy(msg, acc_sh.at[dstloc.at[0]], add=True)
            return carry

        lax.fori_loop(0, n_chunks, chunk, 0)
        plsc.subcore_barrier()
        pltpu.sync_copy(acc_sh.at[pl.ds(s * rows16, rows16)],
                        out_hbm.at[pl.ds(c * NRh + s * rows16, rows16)])

    return k


def _mm_body(x_ref, w_ref, o_ref):
    o_ref[...] = jnp.dot(x_ref[...], w_ref[...],
                         preferred_element_type=jnp.float32)


def _mm(x, w):
    R, K = x.shape
    _, C = w.shape
    Rp = _rup(R, 256)
    Kp = _rup(K, 8)
    Cp = _rup(C, 128)
    xp = jnp.zeros((Rp, Kp), jnp.float32).at[:R, :K].set(x)
    wp = jnp.zeros((Kp, Cp), jnp.float32).at[:K, :C].set(w)
    out = pl.pallas_call(
        _mm_body,
        grid=(Rp // 256,),
        in_specs=[
            pl.BlockSpec((256, Kp), lambda i: (i, 0)),
            pl.BlockSpec((Kp, Cp), lambda i: (0, 0)),
        ],
        out_specs=pl.BlockSpec((256, Cp), lambda i: (i, 0)),
        out_shape=jax.ShapeDtypeStruct((Rp, Cp), jnp.float32),
    )(xp, wp)
    return out[:R, :C]


def _basis16(f):
    outs = []
    for bb in range(8):
        t = jnp.ones((f.shape[0],), jnp.float32)
        for d in range(3):
            fd = f[:, d]
            t = t * (fd if ((bb >> d) & 1) else (1.0 - fd))
        outs.append(t)
    bas = jnp.stack(outs, axis=1)
    return jnp.concatenate(
        [bas, jnp.zeros((f.shape[0], 8), jnp.float32)], axis=1)


def _wflat(W):
    K, ci, co = W.shape
    z = jnp.zeros((8, _L, co), jnp.float32).at[:, :ci, :].set(W)
    return z.reshape(8 * _L, co)


def _pad_tbl(h):
    N, ci = h.shape
    return jnp.zeros((N, _ROW), jnp.float32).at[:, :ci].set(h)


def _bn(x, g, b, eps=1e-5, mask=None, count=None):
    if mask is None:
        m = jnp.mean(x, axis=0)
        v = jnp.var(x, axis=0)
    else:
        xm = jnp.where(mask, x, 0.0)
        m = jnp.sum(xm, axis=0) / count
        d = jnp.where(mask, x - m, 0.0)
        v = jnp.sum(d * d, axis=0) / count
    return (x - m) / jnp.sqrt(v + eps) * g + b


def kernel(x, pos, edge_index, edge_attr, batch, W1, W2, W3, W4, W5, W6, W7,
           g1, g2, g3, g4, g5, g6, g7, b1, b2, b3, b4, b5, b6, b7, fcW):
    src = edge_index[0]
    dst = edge_index[1]
    N = x.shape[0]
    E = src.shape[0]

    basf = _basis16(jnp.clip(edge_attr, 0.0, 1.0))
    ncf = -(-E // (_NS * _CHUNK))
    E_pad = ncf * _NS * _CHUNK
    padn = E_pad - E
    srcp = jnp.concatenate([src, jnp.zeros((padn,), jnp.int32)])
    dstp = jnp.concatenate([dst, jnp.zeros((padn,), jnp.int32)])
    basfp = jnp.concatenate([basf, jnp.zeros((padn, 16), jnp.float32)],
                            axis=0)
    half = _rup((N + 1) // 2 + 1, _CHUNK)
    NRh = _rup(half + 1, _NS * 32)
    zrow = jnp.zeros((32, _ROW), jnp.float32)

    conv = _make_conv(ncf, NRh, half)

    def run_conv(tbl):
        out = conv(tbl, srcp, dstp, basfp, zrow)
        return jnp.concatenate([out[:half], out[NRh:NRh + half]])[:N]

    tbl1 = jnp.zeros((N, _ROW), jnp.float32).at[:, 0].set(x[:, 0])
    tbl1 = tbl1.at[:, 1].set(1.0)
    S = run_conv(tbl1)
    Wf1 = jnp.zeros((8, _L, 9), jnp.float32).at[:, 0, :8].set(W1[:, 0, :])
    Wf1 = Wf1.at[:, 1, 8].set(1.0).reshape(8 * _L, 9)
    agg = _mm(S, Wf1)
    deg = jnp.maximum(agg[:, 8], 1.0)[:, None]
    h = _bn(jax.nn.elu(agg[:, :8] / deg), g1, b1)

    def flayer(h, W, gg, bb):
        S = run_conv(_pad_tbl(h))
        agg = _mm(S, _wflat(W))
        return _bn(jax.nn.elu(agg / deg), gg, bb)

    h = flayer(h, W2, g2, b2)
    sc = h
    h = flayer(h, W3, g3, b3)
    h = flayer(h, W4, g4, b4)
    h = h + sc
    h = flayer(h, W5, g5, b5)

    nx = int(np.ceil(120.0 / 16.0))
    ny = int(np.ceil(100.0 / 12.0))
    cx = jnp.clip(jnp.floor(pos[:, 0] / 16.0), 0, nx - 1).astype(jnp.int32)
    cy = jnp.clip(jnp.floor(pos[:, 1] / 12.0), 0, ny - 1).astype(jnp.int32)
    raw = batch.astype(jnp.int32) * (nx * ny) + cx * ny + cy
    B = 8
    M = B * nx * ny
    px = jax.ops.segment_max(h, raw, num_segments=M)
    cnt = jax.ops.segment_sum(jnp.ones((N,), jnp.float32), raw,
                              num_segments=M)
    ppos = jax.ops.segment_sum(pos, raw, num_segments=M) / jnp.maximum(
        cnt, 1.0)[:, None]
    pbatch = (jnp.arange(M, dtype=jnp.int32) // (nx * ny)).astype(jnp.int32)
    es, ed = raw[src], raw[dst]
    keyk = jnp.where(es != ed, es * M + ed, M * M)
    exists = jnp.zeros((M * M,), bool).at[keyk].set(True, mode='drop')
    ns = (jnp.arange(M * M, dtype=jnp.int32) // M).astype(jnp.int32)
    nd = (jnp.arange(M * M, dtype=jnp.int32) % M).astype(jnp.int32)
    rel = ppos[nd] - ppos[ns]
    a = jnp.where(exists[:, None], jnp.abs(rel), 0.0)
    mmax = jnp.maximum(jnp.max(a), 1e-9)
    ea = rel / (2.0 * mmax) + 0.5
    occ = cnt > 0.0
    nmask = occ[:, None]
    countC = jnp.sum(occ.astype(jnp.float32))

    bas6 = _basis16(jnp.clip(ea, 0.0, 1.0))[:, :8]
    EB = bas6 * exists[:, None].astype(jnp.float32)
    A2 = jnp.transpose(EB.reshape(M, M, 8), (1, 2, 0)).reshape(M, 8 * M)
    deg6 = jnp.maximum(
        jnp.sum(exists.reshape(M, M).astype(jnp.float32), axis=0),
        1.0)[:, None]

    def clayer(hc, W, gg, bb):
        hc = jnp.where(nmask, hc, 0.0)
        xw = _mm(hc, _wcat(W))
        X2 = jnp.transpose(xw.reshape(M, 8, 32), (1, 0, 2)).reshape(8 * M, 32)
        agg = _mm(A2, X2)
        return _bn(jax.nn.elu(agg / deg6), gg, bb, mask=nmask, count=countC)

    sc2 = px
    h2 = clayer(px, W6, g6, b6)
    h2 = clayer(h2, W7, g7, b7)
    h2 = h2 + sc2

    gx = jnp.clip(jnp.floor(ppos[:, 0] / 30.0), 0, 3).astype(jnp.int32)
    gy = jnp.clip(jnp.floor(ppos[:, 1] / 25.0), 0, 3).astype(jnp.int32)
    c7 = jnp.where(occ, pbatch * 16 + gx * 4 + gy, B * 16)
    x16 = jax.ops.segment_max(h2, c7, num_segments=B * 16)
    x16 = jnp.where(jnp.isfinite(x16), x16, 0.0)
    return x16.reshape(B, 16 * 32) @ fcW


def _wcat(W):
    K, ci, co = W.shape
    return jnp.transpose(W, (1, 0, 2)).reshape(ci, K * co)
